# Initial kernel scaffold; baseline (speedup 1.0000x reference)
import jax
import jax.numpy as jnp
from jax.experimental import pallas as pl


def kernel(x_paper, x_author, edge_index_cites, edge_index_writes, edge_index_rev, k_w, k_b, q_w, q_b, v_w, v_b, a_w, a_b, skip, a_rel, m_rel, p_rel, bn_g, bn_b):
    raise NotImplementedError("write your pallas kernel here")


# speedup vs baseline: 21.2298x; 21.2298x over previous
import functools

import jax
import jax.numpy as jnp
from jax.experimental import pallas as pl

_C = 128
_H = 8
_D = 16


def _mm_kernel(x_ref, w_ref, b_ref, o_ref):
    o_ref[...] = (
        jnp.dot(x_ref[...], w_ref[...], preferred_element_type=jnp.float32)
        + b_ref[...]
    )


def _matmul(x, w, b, bm=1024):
    n, c = x.shape
    k = w.shape[1]
    return pl.pallas_call(
        _mm_kernel,
        grid=(pl.cdiv(n, bm),),
        in_specs=[
            pl.BlockSpec((bm, c), lambda i: (i, 0)),
            pl.BlockSpec((c, k), lambda i: (0, 0)),
            pl.BlockSpec((1, k), lambda i: (0, 0)),
        ],
        out_specs=pl.BlockSpec((bm, k), lambda i: (i, 0)),
        out_shape=jax.ShapeDtypeStruct((n, k), jnp.float32),
    )(x, w, b.reshape(1, k))


def _edge_kernel(kg_ref, qg_ref, vg_ref, sp_ref, t_ref, o_ref):
    t = kg_ref[...] * qg_ref[...]
    s = jnp.dot(t, sp_ref[...], preferred_element_type=jnp.float32)
    e = jnp.exp(s)
    e_exp = jnp.dot(e, t_ref[...], preferred_element_type=jnp.float32)
    o_ref[...] = jnp.concatenate([vg_ref[...] * e_exp, e_exp], axis=1)


def _edge_phase(kg, qg, vg, sp, tmat, be=2048):
    e = kg.shape[0]
    return pl.pallas_call(
        _edge_kernel,
        grid=(pl.cdiv(e, be),),
        in_specs=[
            pl.BlockSpec((be, _C), lambda i: (i, 0)),
            pl.BlockSpec((be, _C), lambda i: (i, 0)),
            pl.BlockSpec((be, _C), lambda i: (i, 0)),
            pl.BlockSpec((_C, _H), lambda i: (0, 0)),
            pl.BlockSpec((_H, _C), lambda i: (0, 0)),
        ],
        out_specs=pl.BlockSpec((be, 2 * _C), lambda i: (i, 0)),
        out_shape=jax.ShapeDtypeStruct((e, 2 * _C), jnp.float32),
    )(kg, qg, vg, sp, tmat)


def _c1_kernel(nrows, bm, npairs, g_ref, xp_ref, w_ref, b_ref, s_ref, h_ref, sum_ref, sq_ref):
    agg = g_ref[:, :_C] / (g_ref[:, _C:2 * _C] + 1e-16)
    for p in range(1, npairs):
        o0 = 2 * _C * p
        agg = agg + g_ref[:, o0:o0 + _C] / (g_ref[:, o0 + _C:o0 + 2 * _C] + 1e-16)
    o = jax.nn.gelu(agg)
    o = jnp.dot(o, w_ref[...], preferred_element_type=jnp.float32) + b_ref[...]
    s = s_ref[...]
    h = s * o + (1.0 - s) * xp_ref[...]
    h_ref[...] = h
    i = pl.program_id(0)
    rows = i * bm + jax.lax.broadcasted_iota(jnp.int32, (bm, 1), 0)
    hm = jnp.where(rows < nrows, h, 0.0)
    sum_ref[...] = jnp.sum(hm, axis=0, keepdims=True).reshape(1, 1, _C)
    sq_ref[...] = jnp.sum(hm * hm, axis=0, keepdims=True).reshape(1, 1, _C)


def _post_agg(g, xp, w, b, s_arr, bm=1024):
    n = g.shape[0]
    npairs = g.shape[1] // (2 * _C)
    grid = pl.cdiv(n, bm)
    h, psum, psq = pl.pallas_call(
        functools.partial(_c1_kernel, n, bm, npairs),
        grid=(grid,),
        in_specs=[
            pl.BlockSpec((bm, g.shape[1]), lambda i: (i, 0)),
            pl.BlockSpec((bm, _C), lambda i: (i, 0)),
            pl.BlockSpec((_C, _C), lambda i: (0, 0)),
            pl.BlockSpec((1, _C), lambda i: (0, 0)),
            pl.BlockSpec((1, _C), lambda i: (0, 0)),
        ],
        out_specs=[
            pl.BlockSpec((bm, _C), lambda i: (i, 0)),
            pl.BlockSpec((1, 1, _C), lambda i: (i, 0, 0)),
            pl.BlockSpec((1, 1, _C), lambda i: (i, 0, 0)),
        ],
        out_shape=[
            jax.ShapeDtypeStruct((n, _C), jnp.float32),
            jax.ShapeDtypeStruct((grid, 1, _C), jnp.float32),
            jax.ShapeDtypeStruct((grid, 1, _C), jnp.float32),
        ],
    )(g, xp, w, b.reshape(1, _C), s_arr)
    return h, psum.reshape(grid, _C), psq.reshape(grid, _C)


def _affine_relu_kernel(h_ref, sc_ref, sh_ref, o_ref):
    o_ref[...] = jnp.maximum(h_ref[...] * sc_ref[...] + sh_ref[...], 0.0)


def _affine_relu(h, scale, shift, bm=2048):
    n = h.shape[0]
    return pl.pallas_call(
        _affine_relu_kernel,
        grid=(pl.cdiv(n, bm),),
        in_specs=[
            pl.BlockSpec((bm, _C), lambda i: (i, 0)),
            pl.BlockSpec((1, _C), lambda i: (0, 0)),
            pl.BlockSpec((1, _C), lambda i: (0, 0)),
        ],
        out_specs=pl.BlockSpec((bm, _C), lambda i: (i, 0)),
        out_shape=jax.ShapeDtypeStruct((n, _C), jnp.float32),
    )(h, scale.reshape(1, _C), shift.reshape(1, _C))


def _block_diag(a):
    eye = jnp.eye(_H, dtype=a.dtype)
    return (eye[:, None, :, None] * a[:, :, None, :]).reshape(_C, _C)


def kernel(x_paper, x_author, edge_index_cites, edge_index_writes,
           edge_index_rev, k_w, k_b, q_w, q_b, v_w, v_b, a_w, a_b, skip,
           a_rel, m_rel, p_rel, bn_g, bn_b):
    xs = [x_paper, x_author]
    ns = [x_paper.shape[0], x_author.shape[0]]
    edge_defs = [
        (0, 0, edge_index_cites, 0),
        (1, 0, edge_index_writes, 1),
        (0, 1, edge_index_rev, 2),
    ]
    num_layers = k_w.shape[0]

    ind = (jnp.arange(_C) // _D)[:, None] == jnp.arange(_H)[None, :]
    ind = ind.astype(jnp.float32)
    tmat = ind.T

    inv_sqrt_d = 1.0 / jnp.sqrt(jnp.float32(_D))

    for l in range(num_layers):
        wk, bk, wv, bv = {}, {}, {}, {}
        for r, st in ((0, 0), (1, 1), (2, 0)):
            bd_a = _block_diag(a_rel[l, r])
            bd_m = _block_diag(m_rel[l, r])
            wk[r] = k_w[l, st] @ bd_a
            bk[r] = k_b[l, st] @ bd_a
            wv[r] = v_w[l, st] @ bd_m
            bv[r] = v_b[l, st] @ bd_m

        wcat_p = jnp.concatenate([q_w[l, 0], wk[0], wk[2], wv[0], wv[2]], axis=1)
        bcat_p = jnp.concatenate([q_b[l, 0], bk[0], bk[2], bv[0], bv[2]])
        wcat_a = jnp.concatenate([q_w[l, 1], wk[1], wv[1]], axis=1)
        bcat_a = jnp.concatenate([q_b[l, 1], bk[1], bv[1]])

        proj_p = _matmul(xs[0], wcat_p, bcat_p)
        proj_a = _matmul(xs[1], wcat_a, bcat_a)
        q_nodes = [proj_p[:, :_C], proj_a[:, :_C]]
        ka_nodes = {0: proj_p[:, _C:2 * _C], 2: proj_p[:, 2 * _C:3 * _C],
                    1: proj_a[:, _C:2 * _C]}
        va_nodes = {0: proj_p[:, 3 * _C:4 * _C], 2: proj_p[:, 4 * _C:5 * _C],
                    1: proj_a[:, 2 * _C:3 * _C]}

        agg_parts = [[], []]
        for (st, dt, e_idx, r) in edge_defs:
            src = e_idx[0]
            dst = e_idx[1]
            kg = jnp.take(ka_nodes[r], src, axis=0)
            qg = jnp.take(q_nodes[dt], dst, axis=0)
            vg = jnp.take(va_nodes[r], src, axis=0)
            sp = ind * (p_rel[l, r] * inv_sqrt_d)[None, :]
            ew = _edge_phase(kg, qg, vg, sp, tmat)
            agg_parts[dt].append(
                jax.ops.segment_sum(ew, dst, num_segments=ns[dt]))
        agg = [jnp.concatenate(p, axis=1) if len(p) > 1 else p[0]
               for p in agg_parts]

        new_xs = []
        for t in range(2):
            s_gate = jax.nn.sigmoid(skip[l, t])
            s_arr = jnp.full((1, _C), s_gate, dtype=jnp.float32)
            h, psum, psq = _post_agg(agg[t], xs[t], a_w[l, t], a_b[l, t], s_arr)
            mu = psum.sum(axis=0) / ns[t]
            var = psq.sum(axis=0) / ns[t] - mu * mu
            scale = bn_g[l] / jnp.sqrt(var + 1e-5)
            shift = bn_b[l] - mu * scale
            new_xs.append(_affine_relu(h, scale, shift))
        xs = new_xs

    return xs[0], xs[1]


# speedup vs baseline: 22.4402x; 1.0570x over previous
import functools

import jax
import jax.numpy as jnp
from jax.experimental import pallas as pl

_C = 128
_H = 8
_D = 16


def _mm_kernel(x_ref, w_ref, b_ref, o_ref):
    o_ref[...] = (
        jnp.dot(x_ref[...], w_ref[...], preferred_element_type=jnp.float32)
        + b_ref[...]
    )


def _matmul(x, w, b, bm=1024):
    n, c = x.shape
    k = w.shape[1]
    return pl.pallas_call(
        _mm_kernel,
        grid=(pl.cdiv(n, bm),),
        in_specs=[
            pl.BlockSpec((bm, c), lambda i: (i, 0)),
            pl.BlockSpec((c, k), lambda i: (0, 0)),
            pl.BlockSpec((1, k), lambda i: (0, 0)),
        ],
        out_specs=pl.BlockSpec((bm, k), lambda i: (i, 0)),
        out_shape=jax.ShapeDtypeStruct((n, k), jnp.float32),
    )(x, w, b.reshape(1, k))


def _edge_kernel(kvg_ref, qg_ref, sp_ref, t_ref, w_ref, e_ref):
    t = kvg_ref[:, :_C] * qg_ref[...]
    s = jnp.dot(t, sp_ref[...], preferred_element_type=jnp.float32)
    e = jnp.exp(s)
    e_exp = jnp.dot(e, t_ref[...], preferred_element_type=jnp.float32)
    w_ref[...] = kvg_ref[:, _C:] * e_exp
    e_ref[...] = e


def _edge_phase(kvg, qg, sp, tmat, be=2048):
    e = kvg.shape[0]
    return pl.pallas_call(
        _edge_kernel,
        grid=(pl.cdiv(e, be),),
        in_specs=[
            pl.BlockSpec((be, 2 * _C), lambda i: (i, 0)),
            pl.BlockSpec((be, _C), lambda i: (i, 0)),
            pl.BlockSpec((_C, _H), lambda i: (0, 0)),
            pl.BlockSpec((_H, _C), lambda i: (0, 0)),
        ],
        out_specs=[
            pl.BlockSpec((be, _C), lambda i: (i, 0)),
            pl.BlockSpec((be, _H), lambda i: (i, 0)),
        ],
        out_shape=[
            jax.ShapeDtypeStruct((e, _C), jnp.float32),
            jax.ShapeDtypeStruct((e, _H), jnp.float32),
        ],
    )(kvg, qg, sp, tmat)


def _c1_kernel(nrows, bm, npairs, g_ref, d_ref, t_ref, xp_ref, w_ref, b_ref, s_ref, h_ref, sum_ref, sq_ref):
    tmat = t_ref[...]
    agg = None
    for p in range(npairs):
        den = jnp.dot(d_ref[:, p * _H:(p + 1) * _H], tmat,
                      preferred_element_type=jnp.float32)
        part = g_ref[:, p * _C:(p + 1) * _C] / (den + 1e-16)
        agg = part if agg is None else agg + part
    o = jax.nn.gelu(agg)
    o = jnp.dot(o, w_ref[...], preferred_element_type=jnp.float32) + b_ref[...]
    s = s_ref[...]
    h = s * o + (1.0 - s) * xp_ref[...]
    h_ref[...] = h
    i = pl.program_id(0)
    rows = i * bm + jax.lax.broadcasted_iota(jnp.int32, (bm, 1), 0)
    hm = jnp.where(rows < nrows, h, 0.0)
    sum_ref[...] = jnp.sum(hm, axis=0, keepdims=True).reshape(1, 1, _C)
    sq_ref[...] = jnp.sum(hm * hm, axis=0, keepdims=True).reshape(1, 1, _C)


def _post_agg(g, d, tmat, xp, w, b, s_arr, bm=1024):
    n = g.shape[0]
    npairs = g.shape[1] // _C
    grid = pl.cdiv(n, bm)
    h, psum, psq = pl.pallas_call(
        functools.partial(_c1_kernel, n, bm, npairs),
        grid=(grid,),
        in_specs=[
            pl.BlockSpec((bm, g.shape[1]), lambda i: (i, 0)),
            pl.BlockSpec((bm, d.shape[1]), lambda i: (i, 0)),
            pl.BlockSpec((_H, _C), lambda i: (0, 0)),
            pl.BlockSpec((bm, _C), lambda i: (i, 0)),
            pl.BlockSpec((_C, _C), lambda i: (0, 0)),
            pl.BlockSpec((1, _C), lambda i: (0, 0)),
            pl.BlockSpec((1, _C), lambda i: (0, 0)),
        ],
        out_specs=[
            pl.BlockSpec((bm, _C), lambda i: (i, 0)),
            pl.BlockSpec((1, 1, _C), lambda i: (i, 0, 0)),
            pl.BlockSpec((1, 1, _C), lambda i: (i, 0, 0)),
        ],
        out_shape=[
            jax.ShapeDtypeStruct((n, _C), jnp.float32),
            jax.ShapeDtypeStruct((grid, 1, _C), jnp.float32),
            jax.ShapeDtypeStruct((grid, 1, _C), jnp.float32),
        ],
    )(g, d, tmat, xp, w, b.reshape(1, _C), s_arr)
    return h, psum.reshape(grid, _C), psq.reshape(grid, _C)


def _affine_relu_kernel(h_ref, sc_ref, sh_ref, o_ref):
    o_ref[...] = jnp.maximum(h_ref[...] * sc_ref[...] + sh_ref[...], 0.0)


def _affine_relu(h, scale, shift, bm=2048):
    n = h.shape[0]
    return pl.pallas_call(
        _affine_relu_kernel,
        grid=(pl.cdiv(n, bm),),
        in_specs=[
            pl.BlockSpec((bm, _C), lambda i: (i, 0)),
            pl.BlockSpec((1, _C), lambda i: (0, 0)),
            pl.BlockSpec((1, _C), lambda i: (0, 0)),
        ],
        out_specs=pl.BlockSpec((bm, _C), lambda i: (i, 0)),
        out_shape=jax.ShapeDtypeStruct((n, _C), jnp.float32),
    )(h, scale.reshape(1, _C), shift.reshape(1, _C))


def _block_diag(a):
    eye = jnp.eye(_H, dtype=a.dtype)
    return (eye[:, None, :, None] * a[:, :, None, :]).reshape(_C, _C)


def kernel(x_paper, x_author, edge_index_cites, edge_index_writes,
           edge_index_rev, k_w, k_b, q_w, q_b, v_w, v_b, a_w, a_b, skip,
           a_rel, m_rel, p_rel, bn_g, bn_b):
    xs = [x_paper, x_author]
    ns = [x_paper.shape[0], x_author.shape[0]]
    edge_defs = [
        (0, 0, edge_index_cites, 0),
        (1, 0, edge_index_writes, 1),
        (0, 1, edge_index_rev, 2),
    ]
    num_layers = k_w.shape[0]

    ind = (jnp.arange(_C) // _D)[:, None] == jnp.arange(_H)[None, :]
    ind = ind.astype(jnp.float32)
    tmat = ind.T

    inv_sqrt_d = 1.0 / jnp.sqrt(jnp.float32(_D))

    for l in range(num_layers):
        wk, bk, wv, bv = {}, {}, {}, {}
        for r, st in ((0, 0), (1, 1), (2, 0)):
            bd_a = _block_diag(a_rel[l, r])
            bd_m = _block_diag(m_rel[l, r])
            wk[r] = k_w[l, st] @ bd_a
            bk[r] = k_b[l, st] @ bd_a
            wv[r] = v_w[l, st] @ bd_m
            bv[r] = v_b[l, st] @ bd_m

        wcat_p = jnp.concatenate([q_w[l, 0], wk[0], wv[0], wk[2], wv[2]], axis=1)
        bcat_p = jnp.concatenate([q_b[l, 0], bk[0], bv[0], bk[2], bv[2]])
        wcat_a = jnp.concatenate([q_w[l, 1], wk[1], wv[1]], axis=1)
        bcat_a = jnp.concatenate([q_b[l, 1], bk[1], bv[1]])

        proj_p = _matmul(xs[0], wcat_p, bcat_p)
        proj_a = _matmul(xs[1], wcat_a, bcat_a)
        q_nodes = [proj_p[:, :_C], proj_a[:, :_C]]
        kv_nodes = {0: proj_p[:, _C:3 * _C], 2: proj_p[:, 3 * _C:5 * _C],
                    1: proj_a[:, _C:3 * _C]}

        num_parts = [[], []]
        den_parts = [[], []]
        for (st, dt, e_idx, r) in edge_defs:
            src = e_idx[0]
            dst = e_idx[1]
            kvg = jnp.take(kv_nodes[r], src, axis=0)
            qg = jnp.take(q_nodes[dt], dst, axis=0)
            sp = ind * (p_rel[l, r] * inv_sqrt_d)[None, :]
            w_e, e_e = _edge_phase(kvg, qg, sp, tmat)
            num_parts[dt].append(
                jax.ops.segment_sum(w_e, dst, num_segments=ns[dt]))
            den_parts[dt].append(
                jax.ops.segment_sum(e_e, dst, num_segments=ns[dt]))
        nums = [jnp.concatenate(p, axis=1) if len(p) > 1 else p[0]
                for p in num_parts]
        dens = [jnp.concatenate(p, axis=1) if len(p) > 1 else p[0]
                for p in den_parts]

        new_xs = []
        for t in range(2):
            s_gate = jax.nn.sigmoid(skip[l, t])
            s_arr = jnp.full((1, _C), s_gate, dtype=jnp.float32)
            h, psum, psq = _post_agg(nums[t], dens[t], tmat, xs[t],
                                     a_w[l, t], a_b[l, t], s_arr)
            mu = psum.sum(axis=0) / ns[t]
            var = psq.sum(axis=0) / ns[t] - mu * mu
            scale = bn_g[l] / jnp.sqrt(var + 1e-5)
            shift = bn_b[l] - mu * scale
            new_xs.append(_affine_relu(h, scale, shift))
        xs = new_xs

    return xs[0], xs[1]


# speedup vs baseline: 25.0287x; 1.1154x over previous
import functools

import jax
import jax.numpy as jnp
from jax.experimental import pallas as pl

_C = 128
_H = 8
_D = 16


def _mm_kernel(x_ref, w_ref, b_ref, o_ref):
    o_ref[...] = (
        jnp.dot(x_ref[...], w_ref[...], preferred_element_type=jnp.float32)
        + b_ref[...]
    )


def _matmul(x, w, b, bm=1024):
    n, c = x.shape
    k = w.shape[1]
    return pl.pallas_call(
        _mm_kernel,
        grid=(pl.cdiv(n, bm),),
        in_specs=[
            pl.BlockSpec((bm, c), lambda i: (i, 0)),
            pl.BlockSpec((c, k), lambda i: (0, 0)),
            pl.BlockSpec((1, k), lambda i: (0, 0)),
        ],
        out_specs=pl.BlockSpec((bm, k), lambda i: (i, 0)),
        out_shape=jax.ShapeDtypeStruct((n, k), jnp.float32),
    )(x, w, b.reshape(1, k))


def _edge_kernel(kvg_ref, qg_ref, sp_ref, t_ref, o_ref):
    t = kvg_ref[:, :_C] * qg_ref[...]
    s = jnp.dot(t, sp_ref[...], preferred_element_type=jnp.float32)
    e = jnp.exp(s)
    e_exp = jnp.dot(e, t_ref[...], preferred_element_type=jnp.float32)
    o_ref[...] = jnp.concatenate([kvg_ref[:, _C:] * e_exp, e], axis=1)


def _edge_phase(kvg, qg, sp, tmat, be=2048):
    e = kvg.shape[0]
    return pl.pallas_call(
        _edge_kernel,
        grid=(pl.cdiv(e, be),),
        in_specs=[
            pl.BlockSpec((be, 2 * _C), lambda i: (i, 0)),
            pl.BlockSpec((be, _C), lambda i: (i, 0)),
            pl.BlockSpec((_C, _H), lambda i: (0, 0)),
            pl.BlockSpec((_H, _C), lambda i: (0, 0)),
        ],
        out_specs=pl.BlockSpec((be, _C + _H), lambda i: (i, 0)),
        out_shape=jax.ShapeDtypeStruct((e, _C + _H), jnp.float32),
    )(kvg, qg, sp, tmat)


def _c1_kernel(nrows, bm, npairs, g_ref, t_ref, xp_ref, w_ref, b_ref, s_ref, h_ref, sum_ref, sq_ref):
    tmat = t_ref[...]
    stride = _C + _H
    agg = None
    for p in range(npairs):
        o0 = p * stride
        den = jnp.dot(g_ref[:, o0 + _C:o0 + stride], tmat,
                      preferred_element_type=jnp.float32)
        part = g_ref[:, o0:o0 + _C] / (den + 1e-16)
        agg = part if agg is None else agg + part
    o = jax.nn.gelu(agg)
    o = jnp.dot(o, w_ref[...], preferred_element_type=jnp.float32) + b_ref[...]
    s = s_ref[...]
    h = s * o + (1.0 - s) * xp_ref[...]
    h_ref[...] = h
    i = pl.program_id(0)
    rows = i * bm + jax.lax.broadcasted_iota(jnp.int32, (bm, 1), 0)
    hm = jnp.where(rows < nrows, h, 0.0)
    sum_ref[...] = jnp.sum(hm, axis=0, keepdims=True).reshape(1, 1, _C)
    sq_ref[...] = jnp.sum(hm * hm, axis=0, keepdims=True).reshape(1, 1, _C)


def _post_agg(g, tmat, xp, w, b, s_arr, bm=1024):
    n = g.shape[0]
    npairs = g.shape[1] // (_C + _H)
    grid = pl.cdiv(n, bm)
    h, psum, psq = pl.pallas_call(
        functools.partial(_c1_kernel, n, bm, npairs),
        grid=(grid,),
        in_specs=[
            pl.BlockSpec((bm, g.shape[1]), lambda i: (i, 0)),
            pl.BlockSpec((_H, _C), lambda i: (0, 0)),
            pl.BlockSpec((bm, _C), lambda i: (i, 0)),
            pl.BlockSpec((_C, _C), lambda i: (0, 0)),
            pl.BlockSpec((1, _C), lambda i: (0, 0)),
            pl.BlockSpec((1, _C), lambda i: (0, 0)),
        ],
        out_specs=[
            pl.BlockSpec((bm, _C), lambda i: (i, 0)),
            pl.BlockSpec((1, 1, _C), lambda i: (i, 0, 0)),
            pl.BlockSpec((1, 1, _C), lambda i: (i, 0, 0)),
        ],
        out_shape=[
            jax.ShapeDtypeStruct((n, _C), jnp.float32),
            jax.ShapeDtypeStruct((grid, 1, _C), jnp.float32),
            jax.ShapeDtypeStruct((grid, 1, _C), jnp.float32),
        ],
    )(g, tmat, xp, w, b.reshape(1, _C), s_arr)
    return h, psum.reshape(grid, _C), psq.reshape(grid, _C)


def _affine_relu_kernel(h_ref, sc_ref, sh_ref, o_ref):
    o_ref[...] = jnp.maximum(h_ref[...] * sc_ref[...] + sh_ref[...], 0.0)


def _affine_relu(h, scale, shift, bm=2048):
    n = h.shape[0]
    return pl.pallas_call(
        _affine_relu_kernel,
        grid=(pl.cdiv(n, bm),),
        in_specs=[
            pl.BlockSpec((bm, _C), lambda i: (i, 0)),
            pl.BlockSpec((1, _C), lambda i: (0, 0)),
            pl.BlockSpec((1, _C), lambda i: (0, 0)),
        ],
        out_specs=pl.BlockSpec((bm, _C), lambda i: (i, 0)),
        out_shape=jax.ShapeDtypeStruct((n, _C), jnp.float32),
    )(h, scale.reshape(1, _C), shift.reshape(1, _C))


def _block_diag(a):
    eye = jnp.eye(_H, dtype=a.dtype)
    return (eye[:, None, :, None] * a[:, :, None, :]).reshape(_C, _C)


def kernel(x_paper, x_author, edge_index_cites, edge_index_writes,
           edge_index_rev, k_w, k_b, q_w, q_b, v_w, v_b, a_w, a_b, skip,
           a_rel, m_rel, p_rel, bn_g, bn_b):
    xs = [x_paper, x_author]
    ns = [x_paper.shape[0], x_author.shape[0]]
    edge_defs = [
        (0, 0, edge_index_cites, 0),
        (1, 0, edge_index_writes, 1),
        (0, 1, edge_index_rev, 2),
    ]
    num_layers = k_w.shape[0]

    ind = (jnp.arange(_C) // _D)[:, None] == jnp.arange(_H)[None, :]
    ind = ind.astype(jnp.float32)
    tmat = ind.T

    inv_sqrt_d = 1.0 / jnp.sqrt(jnp.float32(_D))

    for l in range(num_layers):
        wk, bk, wv, bv = {}, {}, {}, {}
        for r, st in ((0, 0), (1, 1), (2, 0)):
            bd_a = _block_diag(a_rel[l, r])
            bd_m = _block_diag(m_rel[l, r])
            wk[r] = k_w[l, st] @ bd_a
            bk[r] = k_b[l, st] @ bd_a
            wv[r] = v_w[l, st] @ bd_m
            bv[r] = v_b[l, st] @ bd_m

        wcat_p = jnp.concatenate([q_w[l, 0], wk[0], wv[0], wk[2], wv[2]], axis=1)
        bcat_p = jnp.concatenate([q_b[l, 0], bk[0], bv[0], bk[2], bv[2]])
        wcat_a = jnp.concatenate([q_w[l, 1], wk[1], wv[1]], axis=1)
        bcat_a = jnp.concatenate([q_b[l, 1], bk[1], bv[1]])

        proj_p = _matmul(xs[0], wcat_p, bcat_p)
        proj_a = _matmul(xs[1], wcat_a, bcat_a)
        q_nodes = [proj_p[:, :_C], proj_a[:, :_C]]
        kv_nodes = {0: proj_p[:, _C:3 * _C], 2: proj_p[:, 3 * _C:5 * _C],
                    1: proj_a[:, _C:3 * _C]}

        agg_parts = [[], []]
        for (st, dt, e_idx, r) in edge_defs:
            src = e_idx[0]
            dst = e_idx[1]
            kvg = jnp.take(kv_nodes[r], src, axis=0)
            qg = jnp.take(q_nodes[dt], dst, axis=0)
            sp = ind * (p_rel[l, r] * inv_sqrt_d)[None, :]
            ew = _edge_phase(kvg, qg, sp, tmat)
            agg_parts[dt].append(
                jax.ops.segment_sum(ew, dst, num_segments=ns[dt]))
        agg = [jnp.concatenate(p, axis=1) if len(p) > 1 else p[0]
               for p in agg_parts]

        new_xs = []
        for t in range(2):
            s_gate = jax.nn.sigmoid(skip[l, t])
            s_arr = jnp.full((1, _C), s_gate, dtype=jnp.float32)
            h, psum, psq = _post_agg(agg[t], tmat, xs[t],
                                     a_w[l, t], a_b[l, t], s_arr)
            mu = psum.sum(axis=0) / ns[t]
            var = psq.sum(axis=0) / ns[t] - mu * mu
            scale = bn_g[l] / jnp.sqrt(var + 1e-5)
            shift = bn_b[l] - mu * scale
            new_xs.append(_affine_relu(h, scale, shift))
        xs = new_xs

    return xs[0], xs[1]
